# baseline (speedup 1.0000x reference)
import jax
import jax.numpy as jnp
import numpy as np
from jax.experimental import pallas as pl

_SUPPORT_P = 0.01
_K2L = float(-np.log(_SUPPORT_P) * np.log2(np.e))

_D = 8
_NB = 64
_S = 1024
_C = 16


def _bump_block_kernel(x_ref, am_ref, bm_ref, mags_ref, y_ref):
    xT = x_ref[:, :].reshape(_S // 16, 16, _D).transpose(2, 0, 1).reshape(_D, _S)
    acc8 = jnp.zeros((8, _S), jnp.float32)
    for c in range(0, _NB, _C):
        qn = jnp.zeros((_C, _S), jnp.float32)
        m = jnp.zeros((_C, _S), jnp.float32)
        for d in range(_D):
            z = xT[d : d + 1, :] * am_ref[c : c + _C, d : d + 1] \
                - bm_ref[c : c + _C, d : d + 1]
            z2 = z * z
            qn = qn - z2
            m = jnp.maximum(m, z2)
        e = jnp.exp2(qn)
        v = mags_ref[c : c + _C, :] * jnp.where(m < _K2L, e, 0.0)
        acc8 = acc8 + v[0:8, :] + v[8:16, :]
    y_ref[:, :] = jnp.sum(acc8, axis=0, keepdims=True)


@jax.jit
def kernel(x, ctrs, band_widths, mags):
    n = x.shape[0]
    npad = -(-n // _S) * _S
    xp = jnp.pad(x, ((0, npad - n), (0, 0))).reshape(npad // 16, 128)

    sql = float(np.sqrt(np.log2(np.e)))
    am = sql / band_widths
    bm = sql * ctrs / band_widths
    mags2 = mags.reshape(_NB, 1)

    grid = (npad // _S,)
    y = pl.pallas_call(
        _bump_block_kernel,
        grid=grid,
        in_specs=[
            pl.BlockSpec((_S // 16, 128), lambda i: (i, 0)),
            pl.BlockSpec((_NB, _D), lambda i: (0, 0)),
            pl.BlockSpec((_NB, _D), lambda i: (0, 0)),
            pl.BlockSpec((_NB, 1), lambda i: (0, 0)),
        ],
        out_specs=pl.BlockSpec((1, _S), lambda i: (0, i)),
        out_shape=jax.ShapeDtypeStruct((1, npad), jnp.float32),
    )(xp, am, bm, mags2)
    return y[0, :n]


# speedup vs baseline: 2.3585x; 2.3585x over previous
import jax
import jax.numpy as jnp
import numpy as np
from jax.experimental import pallas as pl

_SUPPORT_P = 0.01
_K = float(np.sqrt(-np.log(_SUPPORT_P)))
_LOG2E = float(np.log2(np.e))

_D = 8
_NB = 64
_S = 1024
_C = 16


def _bump_block_kernel(xT_ref, w_ref, am_ref, bm_ref, mags_ref, y_ref):
    x = xT_ref[:, :]
    feats = jnp.concatenate(
        [x, x * x, jnp.ones((1, _S), jnp.float32)], axis=0
    )
    earg = jax.lax.dot_general(
        w_ref[:, :], feats,
        dimension_numbers=(((1,), (0,)), ((), ())),
        preferred_element_type=jnp.float32,
        precision=jax.lax.Precision.HIGHEST,
    )
    acc8 = jnp.zeros((8, _S), jnp.float32)
    for c in range(0, _NB, _C):
        m = jnp.zeros((_C, _S), jnp.float32)
        for d in range(_D):
            z = x[d : d + 1, :] * am_ref[c : c + _C, d : d + 1] \
                - bm_ref[c : c + _C, d : d + 1]
            m = jnp.maximum(m, jnp.abs(z))
        e = jnp.exp2(earg[c : c + _C, :])
        v = mags_ref[c : c + _C, :] * jnp.where(m < 1.0, e, 0.0)
        acc8 = acc8 + v[0:8, :] + v[8:16, :]
    y_ref[:, :] = jnp.sum(acc8, axis=0, keepdims=True)


@jax.jit
def kernel(x, ctrs, band_widths, mags):
    n = x.shape[0]
    npad = -(-n // _S) * _S
    xp = jnp.pad(x, ((0, npad - n), (0, 0)))
    xT = xp.T

    a = 1.0 / band_widths
    b = ctrs / band_widths
    w = jnp.concatenate(
        [2.0 * _LOG2E * a * b, -_LOG2E * a * a,
         (-_LOG2E * jnp.sum(b * b, axis=1)).reshape(_NB, 1)],
        axis=1,
    )
    am = a / _K
    bm = b / _K
    mags2 = mags.reshape(_NB, 1)

    grid = (npad // _S,)
    y = pl.pallas_call(
        _bump_block_kernel,
        grid=grid,
        in_specs=[
            pl.BlockSpec((_D, _S), lambda i: (0, i)),
            pl.BlockSpec((_NB, 17), lambda i: (0, 0)),
            pl.BlockSpec((_NB, _D), lambda i: (0, 0)),
            pl.BlockSpec((_NB, _D), lambda i: (0, 0)),
            pl.BlockSpec((_NB, 1), lambda i: (0, 0)),
        ],
        out_specs=pl.BlockSpec((1, _S), lambda i: (0, i)),
        out_shape=jax.ShapeDtypeStruct((1, npad), jnp.float32),
    )(xT, w, am, bm, mags2)
    return y[0, :n]


# speedup vs baseline: 2.4883x; 1.0550x over previous
import jax
import jax.numpy as jnp
import numpy as np
from jax.experimental import pallas as pl

_SUPPORT_P = 0.01
_K = float(np.sqrt(-np.log(_SUPPORT_P)))
_LOG2E = float(np.log2(np.e))

_D = 8
_NB = 64
_S = 1024
_C = 16


def _bump_block_kernel(xT_ref, w_ref, am_ref, bm_ref, mags_ref, y_ref):
    x = xT_ref[:, :]
    feats = jnp.concatenate(
        [x, x * x, jnp.ones((1, _S), jnp.float32)], axis=0
    )
    earg = jax.lax.dot_general(
        w_ref[:, :], feats,
        dimension_numbers=(((1,), (0,)), ((), ())),
        preferred_element_type=jnp.float32,
        precision=jax.lax.Precision.HIGHEST,
    )
    xb = x.astype(jnp.bfloat16)
    acc8 = jnp.zeros((8, _S), jnp.float32)
    for c in range(0, _NB, _C):
        m = jnp.zeros((_C, _S), jnp.bfloat16)
        for d in range(_D):
            z = xb[d : d + 1, :] * am_ref[c : c + _C, d : d + 1] \
                - bm_ref[c : c + _C, d : d + 1]
            m = jnp.maximum(m, jnp.abs(z))
        e = jnp.exp2(earg[c : c + _C, :])
        v = mags_ref[c : c + _C, :] * jnp.where(m < jnp.bfloat16(1.0), e, 0.0)
        acc8 = acc8 + v[0:8, :] + v[8:16, :]
    y_ref[:, :] = jnp.sum(acc8, axis=0, keepdims=True)


@jax.jit
def kernel(x, ctrs, band_widths, mags):
    n = x.shape[0]
    npad = -(-n // _S) * _S
    xp = jnp.pad(x, ((0, npad - n), (0, 0)))
    xT = xp.T

    a = 1.0 / band_widths
    b = ctrs / band_widths
    w = jnp.concatenate(
        [2.0 * _LOG2E * a * b, -_LOG2E * a * a,
         (-_LOG2E * jnp.sum(b * b, axis=1)).reshape(_NB, 1)],
        axis=1,
    )
    am = (a / _K).astype(jnp.bfloat16)
    bm = (b / _K).astype(jnp.bfloat16)
    mags2 = mags.reshape(_NB, 1)

    grid = (npad // _S,)
    y = pl.pallas_call(
        _bump_block_kernel,
        grid=grid,
        in_specs=[
            pl.BlockSpec((_D, _S), lambda i: (0, i)),
            pl.BlockSpec((_NB, 17), lambda i: (0, 0)),
            pl.BlockSpec((_NB, _D), lambda i: (0, 0)),
            pl.BlockSpec((_NB, _D), lambda i: (0, 0)),
            pl.BlockSpec((_NB, 1), lambda i: (0, 0)),
        ],
        out_specs=pl.BlockSpec((1, _S), lambda i: (0, i)),
        out_shape=jax.ShapeDtypeStruct((1, npad), jnp.float32),
    )(xT, w, am, bm, mags2)
    return y[0, :n]


# speedup vs baseline: 3.7697x; 1.5150x over previous
import jax
import jax.numpy as jnp
import numpy as np
from jax.experimental import pallas as pl

_SUPPORT_P = 0.01
_K = float(np.sqrt(-np.log(_SUPPORT_P)))
_LOG2E = float(np.log2(np.e))

_D = 8
_NB = 64
_S = 2048
_C = 16


def _bump_block_kernel(xT_ref, w_ref, am_ref, bm_ref, mags_ref, y_ref):
    x = xT_ref[:, :]
    feats = jnp.concatenate(
        [x, x * x, jnp.ones((1, _S), jnp.float32)], axis=0
    )
    earg = jax.lax.dot_general(
        w_ref[:, :], feats,
        dimension_numbers=(((1,), (0,)), ((), ())),
        preferred_element_type=jnp.float32,
        precision=jax.lax.Precision.HIGHEST,
    )
    xb = x.astype(jnp.bfloat16)
    acc8 = jnp.zeros((8, _S), jnp.float32)
    for c in range(0, _NB, _C):
        m = jnp.zeros((_C, _S), jnp.bfloat16)
        for d in range(_D):
            z = xb[d : d + 1, :] * am_ref[c : c + _C, d : d + 1] \
                - bm_ref[c : c + _C, d : d + 1]
            m = jnp.maximum(m, jnp.abs(z))
        e = jnp.exp2(earg[c : c + _C, :])
        v = mags_ref[c : c + _C, :] * jnp.where(m < jnp.bfloat16(1.0), e, 0.0)
        acc8 = acc8 + v[0:8, :] + v[8:16, :]
    y_ref[:, :] = jnp.sum(acc8, axis=0, keepdims=True)


@jax.jit
def kernel(x, ctrs, band_widths, mags):
    n = x.shape[0]
    npad = -(-n // _S) * _S
    xp = jnp.pad(x, ((0, npad - n), (0, 0)))
    xT = xp.T

    a = 1.0 / band_widths
    b = ctrs / band_widths
    w = jnp.concatenate(
        [2.0 * _LOG2E * a * b, -_LOG2E * a * a,
         (-_LOG2E * jnp.sum(b * b, axis=1)).reshape(_NB, 1)],
        axis=1,
    )
    am = (a / _K).astype(jnp.bfloat16)
    bm = (b / _K).astype(jnp.bfloat16)
    mags2 = mags.reshape(_NB, 1)

    grid = (npad // _S,)
    y = pl.pallas_call(
        _bump_block_kernel,
        grid=grid,
        in_specs=[
            pl.BlockSpec((_D, _S), lambda i: (0, i)),
            pl.BlockSpec((_NB, 17), lambda i: (0, 0)),
            pl.BlockSpec((_NB, _D), lambda i: (0, 0)),
            pl.BlockSpec((_NB, _D), lambda i: (0, 0)),
            pl.BlockSpec((_NB, 1), lambda i: (0, 0)),
        ],
        out_specs=pl.BlockSpec((1, _S), lambda i: (0, i)),
        out_shape=jax.ShapeDtypeStruct((1, npad), jnp.float32),
    )(xT, w, am, bm, mags2)
    return y[0, :n]


# speedup vs baseline: 4.6456x; 1.2324x over previous
import jax
import jax.numpy as jnp
import numpy as np
from jax.experimental import pallas as pl

_SUPPORT_P = 0.01
_K = float(np.sqrt(-np.log(_SUPPORT_P)))
_LOG2E = float(np.log2(np.e))

_D = 8
_NB = 64
_S = 8192
_C = 16


def _bump_block_kernel(xT_ref, w_ref, am_ref, bm_ref, mags_ref, y_ref):
    x = xT_ref[:, :]
    feats = jnp.concatenate(
        [x, x * x, jnp.ones((1, _S), jnp.float32)], axis=0
    )
    earg = jax.lax.dot_general(
        w_ref[:, :], feats,
        dimension_numbers=(((1,), (0,)), ((), ())),
        preferred_element_type=jnp.float32,
        precision=jax.lax.Precision.HIGHEST,
    )
    xb = x.astype(jnp.bfloat16)
    acc8 = jnp.zeros((8, _S), jnp.float32)
    for c in range(0, _NB, _C):
        m = jnp.zeros((_C, _S), jnp.bfloat16)
        for d in range(_D):
            z = xb[d : d + 1, :] * am_ref[c : c + _C, d : d + 1] \
                - bm_ref[c : c + _C, d : d + 1]
            m = jnp.maximum(m, jnp.abs(z))
        e = jnp.exp2(earg[c : c + _C, :])
        v = mags_ref[c : c + _C, :] * jnp.where(m < jnp.bfloat16(1.0), e, 0.0)
        acc8 = acc8 + v[0:8, :] + v[8:16, :]
    y_ref[:, :] = jnp.sum(acc8, axis=0, keepdims=True)


@jax.jit
def kernel(x, ctrs, band_widths, mags):
    n = x.shape[0]
    npad = -(-n // _S) * _S
    xp = jnp.pad(x, ((0, npad - n), (0, 0)))
    xT = xp.T

    a = 1.0 / band_widths
    b = ctrs / band_widths
    w = jnp.concatenate(
        [2.0 * _LOG2E * a * b, -_LOG2E * a * a,
         (-_LOG2E * jnp.sum(b * b, axis=1)).reshape(_NB, 1)],
        axis=1,
    )
    am = (a / _K).astype(jnp.bfloat16)
    bm = (b / _K).astype(jnp.bfloat16)
    mags2 = mags.reshape(_NB, 1)

    grid = (npad // _S,)
    y = pl.pallas_call(
        _bump_block_kernel,
        grid=grid,
        in_specs=[
            pl.BlockSpec((_D, _S), lambda i: (0, i)),
            pl.BlockSpec((_NB, 17), lambda i: (0, 0)),
            pl.BlockSpec((_NB, _D), lambda i: (0, 0)),
            pl.BlockSpec((_NB, _D), lambda i: (0, 0)),
            pl.BlockSpec((_NB, 1), lambda i: (0, 0)),
        ],
        out_specs=pl.BlockSpec((1, _S), lambda i: (0, i)),
        out_shape=jax.ShapeDtypeStruct((1, npad), jnp.float32),
    )(xT, w, am, bm, mags2)
    return y[0, :n]


# speedup vs baseline: 4.6490x; 1.0007x over previous
import jax
import jax.numpy as jnp
import numpy as np
from jax.experimental import pallas as pl

_SUPPORT_P = 0.01
_K = float(np.sqrt(-np.log(_SUPPORT_P)))
_LOG2E = float(np.log2(np.e))

_D = 8
_NB = 64
_S = 16384
_C = 16


def _bump_block_kernel(xT_ref, w_ref, am_ref, bm_ref, mags_ref, y_ref):
    x = xT_ref[:, :]
    feats = jnp.concatenate(
        [x, x * x, jnp.ones((1, _S), jnp.float32)], axis=0
    )
    earg = jax.lax.dot_general(
        w_ref[:, :], feats,
        dimension_numbers=(((1,), (0,)), ((), ())),
        preferred_element_type=jnp.float32,
        precision=jax.lax.Precision.HIGHEST,
    )
    xb = x.astype(jnp.bfloat16)
    acc8 = jnp.zeros((8, _S), jnp.float32)
    for c in range(0, _NB, _C):
        m = jnp.zeros((_C, _S), jnp.bfloat16)
        for d in range(_D):
            z = xb[d : d + 1, :] * am_ref[c : c + _C, d : d + 1] \
                - bm_ref[c : c + _C, d : d + 1]
            m = jnp.maximum(m, jnp.abs(z))
        e = jnp.exp2(earg[c : c + _C, :])
        v = mags_ref[c : c + _C, :] * jnp.where(m < jnp.bfloat16(1.0), e, 0.0)
        acc8 = acc8 + v[0:8, :] + v[8:16, :]
    y_ref[:, :] = jnp.sum(acc8, axis=0, keepdims=True)


@jax.jit
def kernel(x, ctrs, band_widths, mags):
    n = x.shape[0]
    npad = -(-n // _S) * _S
    xp = jnp.pad(x, ((0, npad - n), (0, 0)))
    xT = xp.T

    a = 1.0 / band_widths
    b = ctrs / band_widths
    w = jnp.concatenate(
        [2.0 * _LOG2E * a * b, -_LOG2E * a * a,
         (-_LOG2E * jnp.sum(b * b, axis=1)).reshape(_NB, 1)],
        axis=1,
    )
    am = (a / _K).astype(jnp.bfloat16)
    bm = (b / _K).astype(jnp.bfloat16)
    mags2 = mags.reshape(_NB, 1)

    grid = (npad // _S,)
    y = pl.pallas_call(
        _bump_block_kernel,
        grid=grid,
        in_specs=[
            pl.BlockSpec((_D, _S), lambda i: (0, i)),
            pl.BlockSpec((_NB, 17), lambda i: (0, 0)),
            pl.BlockSpec((_NB, _D), lambda i: (0, 0)),
            pl.BlockSpec((_NB, _D), lambda i: (0, 0)),
            pl.BlockSpec((_NB, 1), lambda i: (0, 0)),
        ],
        out_specs=pl.BlockSpec((1, _S), lambda i: (0, i)),
        out_shape=jax.ShapeDtypeStruct((1, npad), jnp.float32),
    )(xT, w, am, bm, mags2)
    return y[0, :n]
